# Initial kernel scaffold; baseline (speedup 1.0000x reference)
import jax
import jax.numpy as jnp
from jax.experimental import pallas as pl


def kernel(x_mch, x_mft, edge_index, W_mch, b_mch, W_mft, b_mft, Wl1_m2f, bl1_m2f, Wr1_m2f, Wl1_f2m, bl1_f2m, Wr1_f2m, Wl2_m2f, bl2_m2f, Wr2_m2f, Wl2_f2m, bl2_f2m, Wr2_f2m, We1, be1, We2, be2):
    raise NotImplementedError("write your pallas kernel here")


# speedup vs baseline: 4.0282x; 4.0282x over previous
import functools

import jax
import jax.numpy as jnp
from jax import lax
from jax.experimental import pallas as pl
from jax.experimental.pallas import tpu as pltpu
from jax.experimental.pallas import tpu_sc as plsc

N = 10000
E = 320000
H = 64
NC = 2
NS = 16
NW = NC * NS
EPT = E // NW
CH = 80
NCHUNK = EPT // CH
NPAD = 10112
RPS = NPAD // NS

_DOT = functools.partial(jnp.dot, precision=lax.Precision.HIGHEST,
                         preferred_element_type=jnp.float32)


def _tc_a_body(x_m, x_f, Wm, bm, Wf, bf, Wl1m, Wl1f, Wr1m, bl1m, Wr1f, bl1f,
               u1, v1, pf1, pm1):
    h_m = _DOT(x_m[...], Wm[...]) + bm[...]
    h_f = _DOT(x_f[...], Wf[...]) + bf[...]
    u1[...] = _DOT(h_m, Wl1m[...])
    v1[...] = _DOT(h_f, Wl1f[...])
    pf1[...] = _DOT(h_f, Wr1m[...]) + bl1m[...]
    pm1[...] = _DOT(h_m, Wr1f[...]) + bl1f[...]


def _tc_b_body(PF, PM, CF, CM, pf1, pm1, Wl2m, Wl2f, Wr2m, bl2m, Wr2f, bl2f,
               u2, v2, pf2, pm2):
    agg_f = PF[0] + PF[1]
    agg_m = PM[0] + PM[1]
    rf = 1.0 / jnp.maximum(CF[0, :, 0:1] + CF[1, :, 0:1], 1.0)
    rm = 1.0 / jnp.maximum(CM[0, :, 0:1] + CM[1, :, 0:1], 1.0)
    h_f1 = jnp.maximum(agg_f * rf + pf1[...], 0.0)
    h_m1 = jnp.maximum(agg_m * rm + pm1[...], 0.0)
    u2[...] = _DOT(h_m1, Wl2m[...])
    v2[...] = _DOT(h_f1, Wl2f[...])
    pf2[...] = _DOT(h_f1, Wr2m[...]) + bl2m[...]
    pm2[...] = _DOT(h_m1, Wr2f[...]) + bl2f[...]


def _tc_c_body(PF, PM, CF, CM, pf2, pm2, We1t, We1b, be1, p_o, q_o):
    rf = 1.0 / jnp.maximum(CF[0, :, 0:1] + CF[1, :, 0:1], 1.0)
    rm = 1.0 / jnp.maximum(CM[0, :, 0:1] + CM[1, :, 0:1], 1.0)
    h_f2 = (PF[0] + PF[1]) * rf + pf2[...]
    h_m2 = (PM[0] + PM[1]) * rm + pm2[...]
    p_o[...] = _DOT(h_m2, We1t[...])
    q_o[...] = _DOT(h_f2, We1b[...]) + be1[...]


def _row_spec(rb, w):
    return pl.BlockSpec((rb, w), lambda i: (i, 0))


def _full_spec(shape):
    nd = len(shape)
    return pl.BlockSpec(shape, lambda i: (0,) * nd)


def _tc_a(x_m, x_f, Wm, bm, Wf, bf, Wl1m, Wl1f, Wr1m, bl1m, Wr1f, bl1f):
    rb, grid = 2000, 5
    out = [jax.ShapeDtypeStruct((N, H), jnp.float32)] * 4
    return pl.pallas_call(
        _tc_a_body,
        grid=(grid,),
        in_specs=[_row_spec(rb, 128), _row_spec(rb, 128)]
        + [_full_spec(a.shape) for a in (Wm, bm, Wf, bf, Wl1m, Wl1f, Wr1m,
                                         bl1m, Wr1f, bl1f)],
        out_specs=[_row_spec(rb, H)] * 4,
        out_shape=out,
    )(x_m, x_f, Wm, bm, Wf, bf, Wl1m, Wl1f, Wr1m, bl1m, Wr1f, bl1f)


def _tc_b(PF, PM, CF, CM, pf1, pm1, Wl2m, Wl2f, Wr2m, bl2m, Wr2f, bl2f):
    rb, grid = 1000, 10
    part = pl.BlockSpec((2, rb, H), lambda i: (0, i, 0))
    cnt = pl.BlockSpec((2, rb, 16), lambda i: (0, i, 0))
    out = [jax.ShapeDtypeStruct((N, H), jnp.float32)] * 4
    return pl.pallas_call(
        _tc_b_body,
        grid=(grid,),
        in_specs=[part, part, cnt, cnt, _row_spec(rb, H), _row_spec(rb, H)]
        + [_full_spec(a.shape) for a in (Wl2m, Wl2f, Wr2m, bl2m, Wr2f, bl2f)],
        out_specs=[_row_spec(rb, H)] * 4,
        out_shape=out,
    )(PF, PM, CF, CM, pf1, pm1, Wl2m, Wl2f, Wr2m, bl2m, Wr2f, bl2f)


def _tc_c(PF, PM, CF, CM, pf2, pm2, We1t, We1b, be1):
    rb, grid = 1000, 10
    part = pl.BlockSpec((2, rb, H), lambda i: (0, i, 0))
    cnt = pl.BlockSpec((2, rb, 16), lambda i: (0, i, 0))
    out = [jax.ShapeDtypeStruct((N, H), jnp.float32)] * 2
    return pl.pallas_call(
        _tc_c_body,
        grid=(grid,),
        in_specs=[part, part, cnt, cnt, _row_spec(rb, H), _row_spec(rb, H)]
        + [_full_spec(a.shape) for a in (We1t, We1b, be1)],
        out_specs=[_row_spec(rb, H)] * 2,
        out_shape=out,
    )(PF, PM, CF, CM, pf2, pm2, We1t, We1b, be1)


def _sc_mesh():
    return plsc.VectorSubcoreMesh(core_axis_name="c", subcore_axis_name="s",
                                  num_cores=NC, num_subcores=NS)


def _sc_conv_pass(src, dst, u_tab, v_tab, z64, z16, ones, with_counts):
    out_type = [jax.ShapeDtypeStruct((NC, NPAD, H), jnp.float32),
                jax.ShapeDtypeStruct((NC, NPAD, H), jnp.float32)]
    scratch = [
        pltpu.VMEM_SHARED((NPAD, H), jnp.float32),
        pltpu.VMEM_SHARED((NPAD, H), jnp.float32),
        pltpu.VMEM((CH,), jnp.int32),
        pltpu.VMEM((CH,), jnp.int32),
        pltpu.VMEM((CH, H), jnp.float32),
        pltpu.VMEM((CH, H), jnp.float32),
        pltpu.SemaphoreType.DMA,
        pltpu.SemaphoreType.DMA,
    ]
    if with_counts:
        out_type += [jax.ShapeDtypeStruct((NC, NPAD, 16), jnp.float32),
                     jax.ShapeDtypeStruct((NC, NPAD, 16), jnp.float32)]
        scratch += [
            pltpu.VMEM_SHARED((NPAD, 16), jnp.float32),
            pltpu.VMEM_SHARED((NPAD, 16), jnp.float32),
            pltpu.VMEM((CH, 16), jnp.float32),
        ]

    def body(src_h, dst_h, u_h, v_h, z64_h, z16_h, ones_h, pf_h, pm_h,
             *rest):
        if with_counts:
            cf_h, cm_h, accF, accM, idxS, idxD, bufU, bufV, s1, s2, \
                cntF, cntM, ones_v = rest
        else:
            accF, accM, idxS, idxD, bufU, bufV, s1, s2 = rest
        ci = lax.axis_index("c")
        si = lax.axis_index("s")
        wid = ci * NS + si

        r0 = si * RPS
        pltpu.sync_copy(z64_h.at[pl.ds(r0, RPS)], accF.at[pl.ds(r0, RPS)])
        pltpu.sync_copy(z64_h.at[pl.ds(r0, RPS)], accM.at[pl.ds(r0, RPS)])
        if with_counts:
            pltpu.sync_copy(z16_h.at[pl.ds(r0, RPS)], cntF.at[pl.ds(r0, RPS)])
            pltpu.sync_copy(z16_h.at[pl.ds(r0, RPS)], cntM.at[pl.ds(r0, RPS)])
            pltpu.sync_copy(ones_h, ones_v)
        plsc.subcore_barrier()

        base0 = wid * EPT

        @pl.loop(0, NCHUNK)
        def _chunk(i):
            base = base0 + i * CH
            pltpu.sync_copy(src_h.at[pl.ds(base, CH)], idxS)
            pltpu.sync_copy(dst_h.at[pl.ds(base, CH)], idxD)
            cp1 = pltpu.async_copy(u_h.at[idxS], bufU, s1)
            cp2 = pltpu.async_copy(v_h.at[idxD], bufV, s2)
            cp1.wait()
            cp2.wait()
            pltpu.sync_copy(bufU, accF.at[idxD], add=True)
            pltpu.sync_copy(bufV, accM.at[idxS], add=True)
            if with_counts:
                pltpu.sync_copy(ones_v, cntF.at[idxD], add=True)
                pltpu.sync_copy(ones_v, cntM.at[idxS], add=True)

        plsc.subcore_barrier()
        pltpu.sync_copy(accF.at[pl.ds(r0, RPS)], pf_h.at[ci, pl.ds(r0, RPS)])
        pltpu.sync_copy(accM.at[pl.ds(r0, RPS)], pm_h.at[ci, pl.ds(r0, RPS)])
        if with_counts:
            pltpu.sync_copy(cntF.at[pl.ds(r0, RPS)],
                            cf_h.at[ci, pl.ds(r0, RPS)])
            pltpu.sync_copy(cntM.at[pl.ds(r0, RPS)],
                            cm_h.at[ci, pl.ds(r0, RPS)])

    k = pl.kernel(body, out_type=out_type, mesh=_sc_mesh(), scratch_types=scratch,
                  compiler_params=pltpu.CompilerParams(use_tc_tiling_on_sc=False))
    return k(src, dst, u_tab, v_tab, z64, z16, ones)


def _sc_edge_pass(src, dst, p_tab, q_tab, we2, be2):
    out_type = jax.ShapeDtypeStruct((E,), jnp.float32)
    scratch = [
        pltpu.VMEM((CH,), jnp.int32),
        pltpu.VMEM((CH,), jnp.int32),
        pltpu.VMEM((CH, H), jnp.float32),
        pltpu.VMEM((CH, H), jnp.float32),
        pltpu.VMEM((CH,), jnp.float32),
        pltpu.VMEM((H,), jnp.float32),
        pltpu.VMEM((16,), jnp.float32),
        pltpu.SemaphoreType.DMA,
        pltpu.SemaphoreType.DMA,
    ]

    def body(src_h, dst_h, p_h, q_h, w_h, b_h, out_h,
             idxS, idxD, bufP, bufQ, outb, w_sm, b_sm, s1, s2):
        ci = lax.axis_index("c")
        si = lax.axis_index("s")
        wid = ci * NS + si
        pltpu.sync_copy(w_h, w_sm)
        pltpu.sync_copy(b_h, b_sm)
        wvs = [w_sm[pl.ds(16 * j, 16)] for j in range(H // 16)]
        b_s = b_sm[pl.ds(0, 16)][0]
        base0 = wid * EPT
        iota = lax.iota(jnp.int32, 16)

        @pl.loop(0, NCHUNK)
        def _chunk(i):
            base = base0 + i * CH
            pltpu.sync_copy(src_h.at[pl.ds(base, CH)], idxS)
            pltpu.sync_copy(dst_h.at[pl.ds(base, CH)], idxD)
            cp1 = pltpu.async_copy(p_h.at[idxS], bufP, s1)
            cp2 = pltpu.async_copy(q_h.at[idxD], bufQ, s2)
            cp1.wait()
            cp2.wait()
            for g in range(CH // 16):
                rows = iota + (16 * g)
                acc = jnp.zeros((16,), jnp.float32)
                for kk in range(H):
                    cols = jnp.full((16,), kk, jnp.int32)
                    t = (plsc.load_gather(bufP, [rows, cols])
                         + plsc.load_gather(bufQ, [rows, cols]))
                    acc = acc + jnp.maximum(t, 0.0) * wvs[kk // 16][kk % 16]
                z = acc + b_s
                outb[pl.ds(16 * g, 16)] = 1.0 / (1.0 + jnp.exp(-z))
            pltpu.sync_copy(outb, out_h.at[pl.ds(base, CH)])

    k = pl.kernel(body, out_type=out_type, mesh=_sc_mesh(), scratch_types=scratch,
                  compiler_params=pltpu.CompilerParams(use_tc_tiling_on_sc=False,
                                                       needs_layout_passes=False))
    return k(src, dst, p_tab, q_tab, we2, be2)


def kernel(x_mch, x_mft, edge_index, W_mch, b_mch, W_mft, b_mft,
           Wl1_m2f, bl1_m2f, Wr1_m2f, Wl1_f2m, bl1_f2m, Wr1_f2m,
           Wl2_m2f, bl2_m2f, Wr2_m2f, Wl2_f2m, bl2_f2m, Wr2_f2m,
           We1, be1, We2, be2):
    src = edge_index[0].astype(jnp.int32)
    dst = edge_index[1].astype(jnp.int32)

    r2 = lambda b: b.reshape(1, -1)
    z64 = jnp.zeros((NPAD, H), jnp.float32)
    z16 = jnp.zeros((NPAD, 16), jnp.float32)
    ones = jnp.ones((CH, 16), jnp.float32)

    u1, v1, pf1, pm1 = _tc_a(x_mch, x_mft, W_mch, r2(b_mch), W_mft, r2(b_mft),
                             Wl1_m2f, Wl1_f2m, Wr1_m2f, r2(bl1_m2f),
                             Wr1_f2m, r2(bl1_f2m))
    PF, PM, CF, CM = _sc_conv_pass(src, dst, u1, v1, z64, z16, ones, True)
    u2, v2, pf2, pm2 = _tc_b(PF, PM, CF, CM, pf1, pm1,
                             Wl2_m2f, Wl2_f2m, Wr2_m2f, r2(bl2_m2f),
                             Wr2_f2m, r2(bl2_f2m))
    P2F, P2M = _sc_conv_pass(src, dst, u2, v2, z64, z16, ones, False)
    p_tab, q_tab = _tc_c(P2F, P2M, CF, CM, pf2, pm2,
                         We1[:H], We1[H:], r2(be1))
    out = _sc_edge_pass(src, dst, p_tab, q_tab, We2[:, 0],
                        jnp.broadcast_to(be2, (16,)))
    return out.reshape(E, 1)


# speedup vs baseline: 4.6213x; 1.1472x over previous
import functools

import jax
import jax.numpy as jnp
from jax import lax
from jax.experimental import pallas as pl
from jax.experimental.pallas import tpu as pltpu
from jax.experimental.pallas import tpu_sc as plsc

N = 10000
E = 320000
H = 64
NC = 2
NS = 16
NW = NC * NS
EPT = E // NW
CH = 80
NCHUNK = EPT // CH
NPAD = 10112
RPS = NPAD // NS

_DOT = functools.partial(jnp.dot, precision=lax.Precision.HIGHEST,
                         preferred_element_type=jnp.float32)


def _tc_a_body(x_m, x_f, Wm, bm, Wf, bf, Wl1m, Wl1f, Wr1m, bl1m, Wr1f, bl1f,
               u1, v1, pf1, pm1):
    h_m = _DOT(x_m[...], Wm[...]) + bm[...]
    h_f = _DOT(x_f[...], Wf[...]) + bf[...]
    u1[...] = _DOT(h_m, Wl1m[...])
    v1[...] = _DOT(h_f, Wl1f[...])
    pf1[...] = _DOT(h_f, Wr1m[...]) + bl1m[...]
    pm1[...] = _DOT(h_m, Wr1f[...]) + bl1f[...]


def _tc_b_body(PF, PM, CF, CM, pf1, pm1, Wl2m, Wl2f, Wr2m, bl2m, Wr2f, bl2f,
               u2, v2, pf2, pm2):
    agg_f = PF[0] + PF[1]
    agg_m = PM[0] + PM[1]
    rf = 1.0 / jnp.maximum(CF[0, :, 0:1] + CF[1, :, 0:1], 1.0)
    rm = 1.0 / jnp.maximum(CM[0, :, 0:1] + CM[1, :, 0:1], 1.0)
    h_f1 = jnp.maximum(agg_f * rf + pf1[...], 0.0)
    h_m1 = jnp.maximum(agg_m * rm + pm1[...], 0.0)
    u2[...] = _DOT(h_m1, Wl2m[...])
    v2[...] = _DOT(h_f1, Wl2f[...])
    pf2[...] = _DOT(h_f1, Wr2m[...]) + bl2m[...]
    pm2[...] = _DOT(h_m1, Wr2f[...]) + bl2f[...]


def _tc_c_body(PF, PM, CF, CM, pf2, pm2, We1t, We1b, be1, p_o, q_o):
    rf = 1.0 / jnp.maximum(CF[0, :, 0:1] + CF[1, :, 0:1], 1.0)
    rm = 1.0 / jnp.maximum(CM[0, :, 0:1] + CM[1, :, 0:1], 1.0)
    h_f2 = (PF[0] + PF[1]) * rf + pf2[...]
    h_m2 = (PM[0] + PM[1]) * rm + pm2[...]
    p_o[...] = _DOT(h_m2, We1t[...])
    q_o[...] = _DOT(h_f2, We1b[...]) + be1[...]


def _row_spec(rb, w):
    return pl.BlockSpec((rb, w), lambda i: (i, 0))


def _full_spec(shape):
    nd = len(shape)
    return pl.BlockSpec(shape, lambda i: (0,) * nd)


def _tc_a(x_m, x_f, Wm, bm, Wf, bf, Wl1m, Wl1f, Wr1m, bl1m, Wr1f, bl1f):
    rb, grid = 2000, 5
    out = [jax.ShapeDtypeStruct((N, H), jnp.float32)] * 4
    return pl.pallas_call(
        _tc_a_body,
        grid=(grid,),
        in_specs=[_row_spec(rb, 128), _row_spec(rb, 128)]
        + [_full_spec(a.shape) for a in (Wm, bm, Wf, bf, Wl1m, Wl1f, Wr1m,
                                         bl1m, Wr1f, bl1f)],
        out_specs=[_row_spec(rb, H)] * 4,
        out_shape=out,
    )(x_m, x_f, Wm, bm, Wf, bf, Wl1m, Wl1f, Wr1m, bl1m, Wr1f, bl1f)


def _tc_b(PF, PM, CF, CM, pf1, pm1, Wl2m, Wl2f, Wr2m, bl2m, Wr2f, bl2f):
    rb, grid = 1000, 10
    part = pl.BlockSpec((2, rb, H), lambda i: (0, i, 0))
    cnt = pl.BlockSpec((2, rb, 16), lambda i: (0, i, 0))
    out = [jax.ShapeDtypeStruct((N, H), jnp.float32)] * 4
    return pl.pallas_call(
        _tc_b_body,
        grid=(grid,),
        in_specs=[part, part, cnt, cnt, _row_spec(rb, H), _row_spec(rb, H)]
        + [_full_spec(a.shape) for a in (Wl2m, Wl2f, Wr2m, bl2m, Wr2f, bl2f)],
        out_specs=[_row_spec(rb, H)] * 4,
        out_shape=out,
    )(PF, PM, CF, CM, pf1, pm1, Wl2m, Wl2f, Wr2m, bl2m, Wr2f, bl2f)


def _tc_c(PF, PM, CF, CM, pf2, pm2, We1t, We1b, be1):
    rb, grid = 1000, 10
    part = pl.BlockSpec((2, rb, H), lambda i: (0, i, 0))
    cnt = pl.BlockSpec((2, rb, 16), lambda i: (0, i, 0))
    out = [jax.ShapeDtypeStruct((N, H), jnp.float32)] * 2
    return pl.pallas_call(
        _tc_c_body,
        grid=(grid,),
        in_specs=[part, part, cnt, cnt, _row_spec(rb, H), _row_spec(rb, H)]
        + [_full_spec(a.shape) for a in (We1t, We1b, be1)],
        out_specs=[_row_spec(rb, H)] * 2,
        out_shape=out,
    )(PF, PM, CF, CM, pf2, pm2, We1t, We1b, be1)


def _sc_mesh():
    return plsc.VectorSubcoreMesh(core_axis_name="c", subcore_axis_name="s",
                                  num_cores=NC, num_subcores=NS)

_SC_PARAMS = pltpu.CompilerParams(use_tc_tiling_on_sc=False)
_SC_PARAMS_NL = pltpu.CompilerParams(use_tc_tiling_on_sc=False,
                                     needs_layout_passes=False)


def _sc_conv_pass(src, dst, u_tab, v_tab, z64, z16, ones, with_counts):
    out_type = [jax.ShapeDtypeStruct((NC, NPAD, H), jnp.float32),
                jax.ShapeDtypeStruct((NC, NPAD, H), jnp.float32)]
    scratch = [
        pltpu.VMEM_SHARED((NPAD, H), jnp.float32),
        pltpu.VMEM_SHARED((NPAD, H), jnp.float32),
        [pltpu.VMEM((CH,), jnp.int32)] * 2,
        [pltpu.VMEM((CH,), jnp.int32)] * 2,
        [pltpu.VMEM((CH, H), jnp.float32)] * 2,
        [pltpu.VMEM((CH, H), jnp.float32)] * 2,
        [pltpu.SemaphoreType.DMA] * 4,
    ]
    if with_counts:
        out_type += [jax.ShapeDtypeStruct((NC, NPAD, 16), jnp.float32),
                     jax.ShapeDtypeStruct((NC, NPAD, 16), jnp.float32)]
        scratch += [
            pltpu.VMEM_SHARED((NPAD, 16), jnp.float32),
            pltpu.VMEM_SHARED((NPAD, 16), jnp.float32),
            pltpu.VMEM((CH, 16), jnp.float32),
        ]

    def body(src_h, dst_h, u_h, v_h, z64_h, z16_h, ones_h, pf_h, pm_h,
             *rest):
        if with_counts:
            cf_h, cm_h, accF, accM, idxS, idxD, bufU, bufV, sems, \
                cntF, cntM, ones_v = rest
        else:
            accF, accM, idxS, idxD, bufU, bufV, sems = rest
        ci = lax.axis_index("c")
        si = lax.axis_index("s")
        wid = ci * NS + si

        r0 = si * RPS
        pltpu.sync_copy(z64_h.at[pl.ds(r0, RPS)], accF.at[pl.ds(r0, RPS)])
        pltpu.sync_copy(z64_h.at[pl.ds(r0, RPS)], accM.at[pl.ds(r0, RPS)])
        if with_counts:
            pltpu.sync_copy(z16_h.at[pl.ds(r0, RPS)], cntF.at[pl.ds(r0, RPS)])
            pltpu.sync_copy(z16_h.at[pl.ds(r0, RPS)], cntM.at[pl.ds(r0, RPS)])
            pltpu.sync_copy(ones_h, ones_v)
        plsc.subcore_barrier()

        base0 = wid * EPT

        def issue(ch, b):
            base = base0 + ch * CH
            pltpu.sync_copy(src_h.at[pl.ds(base, CH)], idxS[b])
            pltpu.sync_copy(dst_h.at[pl.ds(base, CH)], idxD[b])
            pltpu.async_copy(u_h.at[idxS[b]], bufU[b], sems[2 * b])
            pltpu.async_copy(v_h.at[idxD[b]], bufV[b], sems[2 * b + 1])

        def wait(b):
            pltpu.make_async_copy(u_h.at[idxS[b]], bufU[b], sems[2 * b]).wait()
            pltpu.make_async_copy(v_h.at[idxD[b]], bufV[b],
                                  sems[2 * b + 1]).wait()

        def scatter(b):
            pltpu.sync_copy(bufU[b], accF.at[idxD[b]], add=True)
            pltpu.sync_copy(bufV[b], accM.at[idxS[b]], add=True)
            if with_counts:
                pltpu.sync_copy(ones_v, cntF.at[idxD[b]], add=True)
                pltpu.sync_copy(ones_v, cntM.at[idxS[b]], add=True)

        issue(0, 0)

        @pl.loop(0, (NCHUNK - 1) // 2)
        def _pair(i):
            c0 = 2 * i
            issue(c0 + 1, 1)
            wait(0)
            scatter(0)
            issue(c0 + 2, 0)
            wait(1)
            scatter(1)

        wait(0)
        scatter(0)

        plsc.subcore_barrier()
        pltpu.sync_copy(accF.at[pl.ds(r0, RPS)], pf_h.at[ci, pl.ds(r0, RPS)])
        pltpu.sync_copy(accM.at[pl.ds(r0, RPS)], pm_h.at[ci, pl.ds(r0, RPS)])
        if with_counts:
            pltpu.sync_copy(cntF.at[pl.ds(r0, RPS)],
                            cf_h.at[ci, pl.ds(r0, RPS)])
            pltpu.sync_copy(cntM.at[pl.ds(r0, RPS)],
                            cm_h.at[ci, pl.ds(r0, RPS)])

    k = pl.kernel(body, out_type=out_type, mesh=_sc_mesh(),
                  scratch_types=scratch, compiler_params=_SC_PARAMS)
    return k(src, dst, u_tab, v_tab, z64, z16, ones)


def _sc_edge_pass(src, dst, p_tab, q_tab, wbroad, be2v):
    out_type = jax.ShapeDtypeStruct((E,), jnp.float32)
    NG = CH // 16
    scratch = [
        [pltpu.VMEM((CH,), jnp.int32)] * 2,
        [pltpu.VMEM((CH,), jnp.int32)] * 2,
        [pltpu.VMEM((CH, H), jnp.float32)] * 2,
        [pltpu.VMEM((CH, H), jnp.float32)] * 2,
        pltpu.VMEM((CH,), jnp.float32),
        pltpu.VMEM((H * 16,), jnp.float32),
        pltpu.VMEM((16,), jnp.float32),
        [pltpu.SemaphoreType.DMA] * 4,
    ]

    def body(src_h, dst_h, p_h, q_h, wb_h, be2_h, out_h,
             idxS, idxD, bufP, bufQ, outb, w_sm, b_sm, sems):
        ci = lax.axis_index("c")
        si = lax.axis_index("s")
        wid = ci * NS + si
        pltpu.sync_copy(wb_h, w_sm)
        pltpu.sync_copy(be2_h, b_sm)
        b_s = b_sm[...][0]
        base0 = wid * EPT
        iota = lax.iota(jnp.int32, 16)
        rows = [iota + 16 * g for g in range(NG)]

        def issue(ch, b):
            base = base0 + ch * CH
            pltpu.sync_copy(src_h.at[pl.ds(base, CH)], idxS[b])
            pltpu.sync_copy(dst_h.at[pl.ds(base, CH)], idxD[b])
            pltpu.async_copy(p_h.at[idxS[b]], bufP[b], sems[2 * b])
            pltpu.async_copy(q_h.at[idxD[b]], bufQ[b], sems[2 * b + 1])

        def wait(b):
            pltpu.make_async_copy(p_h.at[idxS[b]], bufP[b], sems[2 * b]).wait()
            pltpu.make_async_copy(q_h.at[idxD[b]], bufQ[b],
                                  sems[2 * b + 1]).wait()

        def compute(ch, b):
            def kstep(kk, accs):
                wb = w_sm[pl.ds(kk * 16, 16)]
                colv = jnp.zeros((16,), jnp.int32) + kk
                new = []
                for g in range(NG):
                    t = (plsc.load_gather(bufP[b], [rows[g], colv])
                         + plsc.load_gather(bufQ[b], [rows[g], colv]))
                    new.append(accs[g] + jnp.maximum(t, 0.0) * wb)
                return tuple(new)

            accs = lax.fori_loop(0, H, kstep,
                                 tuple(jnp.zeros((16,), jnp.float32)
                                       for _ in range(NG)))
            for g in range(NG):
                z = accs[g] + b_s
                outb[pl.ds(16 * g, 16)] = 1.0 / (1.0 + jnp.exp(-z))
            base = base0 + ch * CH
            pltpu.sync_copy(outb, out_h.at[pl.ds(base, CH)])

        issue(0, 0)

        @pl.loop(0, (NCHUNK - 1) // 2)
        def _pair(i):
            c0 = 2 * i
            issue(c0 + 1, 1)
            wait(0)
            compute(c0, 0)
            issue(c0 + 2, 0)
            wait(1)
            compute(c0 + 1, 1)

        wait(0)
        compute(NCHUNK - 1, 0)

    k = pl.kernel(body, out_type=out_type, mesh=_sc_mesh(),
                  scratch_types=scratch, compiler_params=_SC_PARAMS_NL)
    return k(src, dst, p_tab, q_tab, wbroad, be2v)


def kernel(x_mch, x_mft, edge_index, W_mch, b_mch, W_mft, b_mft,
           Wl1_m2f, bl1_m2f, Wr1_m2f, Wl1_f2m, bl1_f2m, Wr1_f2m,
           Wl2_m2f, bl2_m2f, Wr2_m2f, Wl2_f2m, bl2_f2m, Wr2_f2m,
           We1, be1, We2, be2):
    src = edge_index[0].astype(jnp.int32)
    dst = edge_index[1].astype(jnp.int32)

    r2 = lambda b: b.reshape(1, -1)
    z64 = jnp.zeros((NPAD, H), jnp.float32)
    z16 = jnp.zeros((NPAD, 16), jnp.float32)
    ones = jnp.ones((CH, 16), jnp.float32)
    wbroad = jnp.broadcast_to(We2[:, 0:1], (H, 16)).reshape(-1)
    be2v = jnp.broadcast_to(be2, (16,))

    u1, v1, pf1, pm1 = _tc_a(x_mch, x_mft, W_mch, r2(b_mch), W_mft, r2(b_mft),
                             Wl1_m2f, Wl1_f2m, Wr1_m2f, r2(bl1_m2f),
                             Wr1_f2m, r2(bl1_f2m))
    PF, PM, CF, CM = _sc_conv_pass(src, dst, u1, v1, z64, z16, ones, True)
    u2, v2, pf2, pm2 = _tc_b(PF, PM, CF, CM, pf1, pm1,
                             Wl2_m2f, Wl2_f2m, Wr2_m2f, r2(bl2_m2f),
                             Wr2_f2m, r2(bl2_f2m))
    P2F, P2M = _sc_conv_pass(src, dst, u2, v2, z64, z16, ones, False)
    p_tab, q_tab = _tc_c(P2F, P2M, CF, CM, pf2, pm2,
                         We1[:H], We1[H:], r2(be1))
    out = _sc_edge_pass(src, dst, p_tab, q_tab, wbroad, be2v)
    return out.reshape(E, 1)


# speedup vs baseline: 7.2927x; 1.5781x over previous
import functools

import jax
import jax.numpy as jnp
from jax import lax
from jax.experimental import pallas as pl
from jax.experimental.pallas import tpu as pltpu
from jax.experimental.pallas import tpu_sc as plsc

N = 10000
E = 320000
H = 64
NC = 2
NS = 16
NW = NC * NS
EPT = E // NW
CH = 80
NCHUNK = EPT // CH
NPAD = 10112
RPS = NPAD // NS

_DOT = functools.partial(jnp.dot, precision=lax.Precision.HIGHEST,
                         preferred_element_type=jnp.float32)


def _tc_a_body(x_m, x_f, Wm, bm, Wf, bf, Wl1m, Wl1f, Wr1m, bl1m, Wr1f, bl1f,
               u1, v1, pf1, pm1):
    h_m = _DOT(x_m[...], Wm[...]) + bm[...]
    h_f = _DOT(x_f[...], Wf[...]) + bf[...]
    u1[...] = _DOT(h_m, Wl1m[...])
    v1[...] = _DOT(h_f, Wl1f[...])
    pf1[...] = _DOT(h_f, Wr1m[...]) + bl1m[...]
    pm1[...] = _DOT(h_m, Wr1f[...]) + bl1f[...]


def _tc_b_body(PF, PM, CF, CM, pf1, pm1, Wl2m, Wl2f, Wr2m, bl2m, Wr2f, bl2f,
               u2, v2, pf2, pm2):
    agg_f = PF[0] + PF[1]
    agg_m = PM[0] + PM[1]
    rf = 1.0 / jnp.maximum(CF[0, :, 0:1] + CF[1, :, 0:1], 1.0)
    rm = 1.0 / jnp.maximum(CM[0, :, 0:1] + CM[1, :, 0:1], 1.0)
    h_f1 = jnp.maximum(agg_f * rf + pf1[...], 0.0)
    h_m1 = jnp.maximum(agg_m * rm + pm1[...], 0.0)
    u2[...] = _DOT(h_m1, Wl2m[...])
    v2[...] = _DOT(h_f1, Wl2f[...])
    pf2[...] = _DOT(h_f1, Wr2m[...]) + bl2m[...]
    pm2[...] = _DOT(h_m1, Wr2f[...]) + bl2f[...]


def _tc_c_body(PF, PM, CF, CM, pf2, pm2, We1t, We1b, be1, p_o, q_o):
    rf = 1.0 / jnp.maximum(CF[0, :, 0:1] + CF[1, :, 0:1], 1.0)
    rm = 1.0 / jnp.maximum(CM[0, :, 0:1] + CM[1, :, 0:1], 1.0)
    h_f2 = (PF[0] + PF[1]) * rf + pf2[...]
    h_m2 = (PM[0] + PM[1]) * rm + pm2[...]
    p_o[...] = _DOT(h_m2, We1t[...])
    q_o[...] = _DOT(h_f2, We1b[...]) + be1[...]


def _row_spec(rb, w):
    return pl.BlockSpec((rb, w), lambda i: (i, 0))


def _full_spec(shape):
    nd = len(shape)
    return pl.BlockSpec(shape, lambda i: (0,) * nd)


def _tc_a(x_m, x_f, Wm, bm, Wf, bf, Wl1m, Wl1f, Wr1m, bl1m, Wr1f, bl1f):
    rb, grid = 2000, 5
    out = [jax.ShapeDtypeStruct((N, H), jnp.float32)] * 4
    return pl.pallas_call(
        _tc_a_body,
        grid=(grid,),
        in_specs=[_row_spec(rb, 128), _row_spec(rb, 128)]
        + [_full_spec(a.shape) for a in (Wm, bm, Wf, bf, Wl1m, Wl1f, Wr1m,
                                         bl1m, Wr1f, bl1f)],
        out_specs=[_row_spec(rb, H)] * 4,
        out_shape=out,
    )(x_m, x_f, Wm, bm, Wf, bf, Wl1m, Wl1f, Wr1m, bl1m, Wr1f, bl1f)


def _tc_b(PF, PM, CF, CM, pf1, pm1, Wl2m, Wl2f, Wr2m, bl2m, Wr2f, bl2f):
    rb, grid = 1000, 10
    part = pl.BlockSpec((2, rb, H), lambda i: (0, i, 0))
    cnt = pl.BlockSpec((2, rb, 16), lambda i: (0, i, 0))
    out = [jax.ShapeDtypeStruct((N, H), jnp.float32)] * 4
    return pl.pallas_call(
        _tc_b_body,
        grid=(grid,),
        in_specs=[part, part, cnt, cnt, _row_spec(rb, H), _row_spec(rb, H)]
        + [_full_spec(a.shape) for a in (Wl2m, Wl2f, Wr2m, bl2m, Wr2f, bl2f)],
        out_specs=[_row_spec(rb, H)] * 4,
        out_shape=out,
    )(PF, PM, CF, CM, pf1, pm1, Wl2m, Wl2f, Wr2m, bl2m, Wr2f, bl2f)


def _tc_c(PF, PM, CF, CM, pf2, pm2, We1t, We1b, be1):
    rb, grid = 1000, 10
    part = pl.BlockSpec((2, rb, H), lambda i: (0, i, 0))
    cnt = pl.BlockSpec((2, rb, 16), lambda i: (0, i, 0))
    out = [jax.ShapeDtypeStruct((N, H), jnp.float32)] * 2
    return pl.pallas_call(
        _tc_c_body,
        grid=(grid,),
        in_specs=[part, part, cnt, cnt, _row_spec(rb, H), _row_spec(rb, H)]
        + [_full_spec(a.shape) for a in (We1t, We1b, be1)],
        out_specs=[_row_spec(rb, H)] * 2,
        out_shape=out,
    )(PF, PM, CF, CM, pf2, pm2, We1t, We1b, be1)


def _sc_mesh():
    return plsc.VectorSubcoreMesh(core_axis_name="c", subcore_axis_name="s",
                                  num_cores=NC, num_subcores=NS)

_SC_PARAMS = pltpu.CompilerParams(use_tc_tiling_on_sc=False)
_SC_PARAMS_NL = pltpu.CompilerParams(use_tc_tiling_on_sc=False,
                                     needs_layout_passes=False)


def _sc_conv_pass(src, dst, u_tab, v_tab, z64, z16, ones, with_counts):
    out_type = [jax.ShapeDtypeStruct((NC, NPAD, H), jnp.float32),
                jax.ShapeDtypeStruct((NC, NPAD, H), jnp.float32)]
    scratch = [
        pltpu.VMEM_SHARED((NPAD, H), jnp.float32),
        pltpu.VMEM_SHARED((NPAD, H), jnp.float32),
        [pltpu.VMEM((CH,), jnp.int32)] * 2,
        [pltpu.VMEM((CH,), jnp.int32)] * 2,
        [pltpu.VMEM((CH, H), jnp.float32)] * 2,
        [pltpu.VMEM((CH, H), jnp.float32)] * 2,
        [pltpu.SemaphoreType.DMA] * 4,
    ]
    if with_counts:
        out_type += [jax.ShapeDtypeStruct((NC, NPAD, 16), jnp.float32),
                     jax.ShapeDtypeStruct((NC, NPAD, 16), jnp.float32)]
        scratch += [
            pltpu.VMEM_SHARED((NPAD, 16), jnp.float32),
            pltpu.VMEM_SHARED((NPAD, 16), jnp.float32),
            pltpu.VMEM((CH, 16), jnp.float32),
        ]

    def body(src_h, dst_h, u_h, v_h, z64_h, z16_h, ones_h, pf_h, pm_h,
             *rest):
        if with_counts:
            cf_h, cm_h, accF, accM, idxS, idxD, bufU, bufV, sems, \
                cntF, cntM, ones_v = rest
        else:
            accF, accM, idxS, idxD, bufU, bufV, sems = rest
        ci = lax.axis_index("c")
        si = lax.axis_index("s")
        wid = ci * NS + si

        r0 = si * RPS
        pltpu.sync_copy(z64_h.at[pl.ds(r0, RPS)], accF.at[pl.ds(r0, RPS)])
        pltpu.sync_copy(z64_h.at[pl.ds(r0, RPS)], accM.at[pl.ds(r0, RPS)])
        if with_counts:
            pltpu.sync_copy(z16_h.at[pl.ds(r0, RPS)], cntF.at[pl.ds(r0, RPS)])
            pltpu.sync_copy(z16_h.at[pl.ds(r0, RPS)], cntM.at[pl.ds(r0, RPS)])
            pltpu.sync_copy(ones_h, ones_v)
        plsc.subcore_barrier()

        base0 = wid * EPT

        def issue(ch, b):
            base = base0 + ch * CH
            pltpu.sync_copy(src_h.at[pl.ds(base, CH)], idxS[b])
            pltpu.sync_copy(dst_h.at[pl.ds(base, CH)], idxD[b])
            pltpu.async_copy(u_h.at[idxS[b]], bufU[b], sems[2 * b])
            pltpu.async_copy(v_h.at[idxD[b]], bufV[b], sems[2 * b + 1])

        def wait(b):
            pltpu.make_async_copy(u_h.at[idxS[b]], bufU[b], sems[2 * b]).wait()
            pltpu.make_async_copy(v_h.at[idxD[b]], bufV[b],
                                  sems[2 * b + 1]).wait()

        def scatter(b):
            pltpu.sync_copy(bufU[b], accF.at[idxD[b]], add=True)
            pltpu.sync_copy(bufV[b], accM.at[idxS[b]], add=True)
            if with_counts:
                pltpu.sync_copy(ones_v, cntF.at[idxD[b]], add=True)
                pltpu.sync_copy(ones_v, cntM.at[idxS[b]], add=True)

        issue(0, 0)

        @pl.loop(0, (NCHUNK - 1) // 2)
        def _pair(i):
            c0 = 2 * i
            issue(c0 + 1, 1)
            wait(0)
            scatter(0)
            issue(c0 + 2, 0)
            wait(1)
            scatter(1)

        wait(0)
        scatter(0)

        plsc.subcore_barrier()
        pltpu.sync_copy(accF.at[pl.ds(r0, RPS)], pf_h.at[ci, pl.ds(r0, RPS)])
        pltpu.sync_copy(accM.at[pl.ds(r0, RPS)], pm_h.at[ci, pl.ds(r0, RPS)])
        if with_counts:
            pltpu.sync_copy(cntF.at[pl.ds(r0, RPS)],
                            cf_h.at[ci, pl.ds(r0, RPS)])
            pltpu.sync_copy(cntM.at[pl.ds(r0, RPS)],
                            cm_h.at[ci, pl.ds(r0, RPS)])

    k = pl.kernel(body, out_type=out_type, mesh=_sc_mesh(),
                  scratch_types=scratch, compiler_params=_SC_PARAMS)
    return k(src, dst, u_tab, v_tab, z64, z16, ones)


def _sc_edge_pass(src, dst, p_tab, q_tab, we2v, be2v):
    out_type = jax.ShapeDtypeStruct((E,), jnp.float32)
    NG = CH // 16
    scratch = [
        [pltpu.VMEM((CH,), jnp.int32)] * 2,
        [pltpu.VMEM((CH,), jnp.int32)] * 2,
        [pltpu.VMEM((CH, H), jnp.float32)] * 2,
        [pltpu.VMEM((CH, H), jnp.float32)] * 2,
        pltpu.VMEM((CH,), jnp.float32),
        pltpu.VMEM((H,), jnp.float32),
        pltpu.VMEM((16,), jnp.float32),
        [pltpu.SemaphoreType.DMA] * 4,
    ]

    def body(src_h, dst_h, p_h, q_h, w_h, be2_h, out_h,
             idxS, idxD, bufP, bufQ, outb, w_sm, b_sm, sems):
        ci = lax.axis_index("c")
        si = lax.axis_index("s")
        wid = ci * NS + si
        pltpu.sync_copy(w_h, w_sm)
        pltpu.sync_copy(be2_h, b_sm)
        b_s = b_sm[...][0]
        base0 = wid * EPT
        iota = lax.iota(jnp.int32, 16)

        def issue(ch, b):
            base = base0 + ch * CH
            pltpu.sync_copy(src_h.at[pl.ds(base, CH)], idxS[b])
            pltpu.sync_copy(dst_h.at[pl.ds(base, CH)], idxD[b])
            pltpu.async_copy(p_h.at[idxS[b]], bufP[b], sems[2 * b])
            pltpu.async_copy(q_h.at[idxD[b]], bufQ[b], sems[2 * b + 1])

        def wait(b):
            pltpu.make_async_copy(p_h.at[idxS[b]], bufP[b], sems[2 * b]).wait()
            pltpu.make_async_copy(q_h.at[idxD[b]], bufQ[b],
                                  sems[2 * b + 1]).wait()

        wvs = [w_sm[pl.ds(16 * j, 16)] for j in range(H // 16)]
        mask15 = iota == 15
        zi16 = jnp.zeros((16,), jnp.int32)

        def compute(ch, b):
            @pl.loop(0, CH)
            def _edge(e):
                acc = None
                for j in range(H // 16):
                    t = jnp.maximum(bufP[b][e, pl.ds(16 * j, 16)]
                                    + bufQ[b][e, pl.ds(16 * j, 16)],
                                    0.0) * wvs[j]
                    acc = t if acc is None else acc + t
                cs = plsc.cumsum(acc)
                plsc.store_scatter(outb, [zi16 + e], cs, mask=mask15)
            for g in range(NG):
                z = outb[pl.ds(16 * g, 16)] + b_s
                outb[pl.ds(16 * g, 16)] = 1.0 / (1.0 + jnp.exp(-z))
            base = base0 + ch * CH
            pltpu.sync_copy(outb, out_h.at[pl.ds(base, CH)])

        issue(0, 0)

        @pl.loop(0, (NCHUNK - 1) // 2)
        def _pair(i):
            c0 = 2 * i
            issue(c0 + 1, 1)
            wait(0)
            compute(c0, 0)
            issue(c0 + 2, 0)
            wait(1)
            compute(c0 + 1, 1)

        wait(0)
        compute(NCHUNK - 1, 0)

    k = pl.kernel(body, out_type=out_type, mesh=_sc_mesh(),
                  scratch_types=scratch, compiler_params=_SC_PARAMS_NL)
    return k(src, dst, p_tab, q_tab, we2v, be2v)


def kernel(x_mch, x_mft, edge_index, W_mch, b_mch, W_mft, b_mft,
           Wl1_m2f, bl1_m2f, Wr1_m2f, Wl1_f2m, bl1_f2m, Wr1_f2m,
           Wl2_m2f, bl2_m2f, Wr2_m2f, Wl2_f2m, bl2_f2m, Wr2_f2m,
           We1, be1, We2, be2):
    src = edge_index[0].astype(jnp.int32)
    dst = edge_index[1].astype(jnp.int32)

    r2 = lambda b: b.reshape(1, -1)
    z64 = jnp.zeros((NPAD, H), jnp.float32)
    z16 = jnp.zeros((NPAD, 16), jnp.float32)
    ones = jnp.ones((CH, 16), jnp.float32)
    be2v = jnp.broadcast_to(be2, (16,))

    u1, v1, pf1, pm1 = _tc_a(x_mch, x_mft, W_mch, r2(b_mch), W_mft, r2(b_mft),
                             Wl1_m2f, Wl1_f2m, Wr1_m2f, r2(bl1_m2f),
                             Wr1_f2m, r2(bl1_f2m))
    PF, PM, CF, CM = _sc_conv_pass(src, dst, u1, v1, z64, z16, ones, True)
    u2, v2, pf2, pm2 = _tc_b(PF, PM, CF, CM, pf1, pm1,
                             Wl2_m2f, Wl2_f2m, Wr2_m2f, r2(bl2_m2f),
                             Wr2_f2m, r2(bl2_f2m))
    P2F, P2M = _sc_conv_pass(src, dst, u2, v2, z64, z16, ones, False)
    p_tab, q_tab = _tc_c(P2F, P2M, CF, CM, pf2, pm2,
                         We1[:H], We1[H:], r2(be1))
    out = _sc_edge_pass(src, dst, p_tab, q_tab, We2[:, 0], be2v)
    return out.reshape(E, 1)
